# baseline (speedup 1.0000x reference)
import functools

import jax
import jax.numpy as jnp
from jax import lax
from jax.experimental import pallas as pl
from jax.experimental.pallas import tpu as pltpu
from jax.experimental.pallas import tpu_sc as plsc

EMB = 64
LANES = 16
NUM_WORKERS = 32
CHUNK = 512
GATHER = 128


def _body(inp_hbm, lab_hbm, tab_hbm, out_hbm, idx_v, rows_v, inp_v, sem_g,
          sem_i, *, rows_per_worker, num_chunks):
    wid = lax.axis_index("s") * 2 + lax.axis_index("c")
    base0 = wid * rows_per_worker

    def chunk_body(i, carry):
        base = base0 + i * CHUNK
        pltpu.sync_copy(lab_hbm.at[pl.ds(base, CHUNK)], idx_v)
        cp_in = pltpu.async_copy(inp_hbm.at[pl.ds(base, CHUNK)], inp_v, sem_i)
        gathers = []
        for g in range(CHUNK // GATHER):
            sl = pl.ds(g * GATHER, GATHER)
            gathers.append(
                pltpu.async_copy(tab_hbm.at[idx_v.at[sl]], rows_v.at[sl],
                                 sem_g))
        for cp in gathers:
            cp.wait()
        cp_in.wait()

        def add_row(r, c):
            for j in range(EMB // LANES):
                sl = pl.ds(j * LANES, LANES)
                rows_v[r, sl] = rows_v[r, sl] + inp_v[r, sl]
            return c

        lax.fori_loop(0, CHUNK, add_row, 0, unroll=2)
        pltpu.sync_copy(rows_v, out_hbm.at[pl.ds(base, CHUNK)])
        return carry

    lax.fori_loop(0, num_chunks, chunk_body, 0)


def kernel(inputs, labels, emb_table):
    batch, hist, emb = inputs.shape
    rows = batch * hist
    assert emb == EMB and rows % (NUM_WORKERS * CHUNK) == 0
    rows_per_worker = rows // NUM_WORKERS
    num_chunks = rows_per_worker // CHUNK

    inp2d = inputs.reshape(rows, emb)
    lab1d = labels.reshape(rows).astype(jnp.int32)

    mesh = plsc.VectorSubcoreMesh(core_axis_name="c", subcore_axis_name="s")
    run = pl.kernel(
        functools.partial(_body, rows_per_worker=rows_per_worker,
                          num_chunks=num_chunks),
        out_type=jax.ShapeDtypeStruct((rows, emb), jnp.float32),
        mesh=mesh,
        scratch_types=[
            pltpu.VMEM((CHUNK,), jnp.int32),
            pltpu.VMEM((CHUNK, EMB), jnp.float32),
            pltpu.VMEM((CHUNK, EMB), jnp.float32),
            pltpu.SemaphoreType.DMA,
            pltpu.SemaphoreType.DMA,
        ],
        compiler_params=pltpu.CompilerParams(use_tc_tiling_on_sc=False),
    )
    out = run(inp2d, lab1d, emb_table)
    return out.reshape(batch, hist, emb)


# speedup vs baseline: 1.3074x; 1.3074x over previous
import functools

import jax
import jax.numpy as jnp
from jax import lax
from jax.experimental import pallas as pl
from jax.experimental.pallas import tpu as pltpu
from jax.experimental.pallas import tpu_sc as plsc

EMB = 64
NUM_WORKERS = 32
CHUNK = 512
GATHER = 128


def _body(inp_hbm, lab_hbm, tab_hbm, out_hbm, idx_v, buf_v, si, sp, sg, so,
          *, rows_per_worker, num_chunks):
    wid = lax.axis_index("s") * 2 + lax.axis_index("c")
    base0 = wid * rows_per_worker

    def issue_loads(i, p):
        base = base0 + i * CHUNK
        pltpu.async_copy(lab_hbm.at[pl.ds(base, CHUNK)], idx_v.at[p], si)
        pltpu.async_copy(inp_hbm.at[pl.ds(base, CHUNK)], buf_v.at[p], sp)

    def wait_loads(i, p):
        base = base0 + i * CHUNK
        pltpu.make_async_copy(lab_hbm.at[pl.ds(base, CHUNK)], idx_v.at[p],
                              si).wait()
        pltpu.make_async_copy(inp_hbm.at[pl.ds(base, CHUNK)], buf_v.at[p],
                              sp).wait()

    issue_loads(0, 0)

    def chunk_body(i, carry):
        p = lax.rem(i, 2)
        pn = 1 - p
        wait_loads(i, p)
        for g in range(CHUNK // GATHER):
            sl = pl.ds(g * GATHER, GATHER)
            pltpu.async_copy(tab_hbm.at[idx_v.at[p].at[sl]],
                             buf_v.at[p].at[sl], sg, add=True)
        @pl.when(i >= 1)
        def _():
            basep = base0 + (i - 1) * CHUNK
            pltpu.make_async_copy(buf_v.at[pn],
                                  out_hbm.at[pl.ds(basep, CHUNK)], so).wait()

        @pl.when(i + 1 < num_chunks)
        def _():
            issue_loads(i + 1, pn)

        for g in range(CHUNK // GATHER):
            sl = pl.ds(g * GATHER, GATHER)
            pltpu.make_async_copy(tab_hbm.at[idx_v.at[p].at[sl]],
                                  buf_v.at[p].at[sl], sg).wait()
        base = base0 + i * CHUNK
        pltpu.async_copy(buf_v.at[p], out_hbm.at[pl.ds(base, CHUNK)], so)
        return carry

    lax.fori_loop(0, num_chunks, chunk_body, 0)
    pl_last = lax.rem(num_chunks - 1, 2)
    base_last = base0 + (num_chunks - 1) * CHUNK
    pltpu.make_async_copy(buf_v.at[pl_last],
                          out_hbm.at[pl.ds(base_last, CHUNK)], so).wait()


def kernel(inputs, labels, emb_table):
    batch, hist, emb = inputs.shape
    rows = batch * hist
    assert emb == EMB and rows % (NUM_WORKERS * CHUNK) == 0
    rows_per_worker = rows // NUM_WORKERS
    num_chunks = rows_per_worker // CHUNK

    inp2d = inputs.reshape(rows, emb)
    lab1d = labels.reshape(rows).astype(jnp.int32)

    mesh = plsc.VectorSubcoreMesh(core_axis_name="c", subcore_axis_name="s")
    run = pl.kernel(
        functools.partial(_body, rows_per_worker=rows_per_worker,
                          num_chunks=num_chunks),
        out_type=jax.ShapeDtypeStruct((rows, emb), jnp.float32),
        mesh=mesh,
        scratch_types=[
            pltpu.VMEM((2, CHUNK), jnp.int32),
            pltpu.VMEM((2, CHUNK, EMB), jnp.float32),
            pltpu.SemaphoreType.DMA,
            pltpu.SemaphoreType.DMA,
            pltpu.SemaphoreType.DMA,
            pltpu.SemaphoreType.DMA,
        ],
        compiler_params=pltpu.CompilerParams(use_tc_tiling_on_sc=False),
    )
    out = run(inp2d, lab1d, emb_table)
    return out.reshape(batch, hist, emb)
